# baseline (speedup 1.0000x reference)
import functools

import jax
import jax.numpy as jnp
from jax import lax
from jax.experimental import pallas as pl
from jax.experimental.pallas import tpu as pltpu
from jax.experimental.pallas import tpu_sc as plsc

BATCH = 16384
EMBED = 128
NC, NS = 2, 16
NW = NC * NS
CHUNK = 128
NSPLIT = 2


def _sc_gather4(users, items, t_ug, t_ig, t_um, t_im, si, n):
    b_per_w = n // NW
    assert b_per_w == 2 * CHUNK
    mesh = plsc.VectorSubcoreMesh(core_axis_name="c", subcore_axis_name="s")
    row_t = jax.ShapeDtypeStruct((n, EMBED), jnp.float32)

    @functools.partial(
        pl.kernel,
        mesh=mesh,
        out_type=(row_t, row_t, row_t),
        scratch_types=[
            pltpu.VMEM((b_per_w,), jnp.int32),
            pltpu.VMEM((b_per_w,), jnp.int32),
            pltpu.VMEM((2, CHUNK, EMBED), jnp.float32),
            pltpu.VMEM((2, CHUNK, EMBED), jnp.float32),
            pltpu.VMEM((3, CHUNK, EMBED), jnp.float32),
        ] + [pltpu.SemaphoreType.DMA] * 14,
    )
    def k(u_ref, i_ref, ug_ref, ig_ref, um_ref, im_ref,
          o_gp, o_um, o_im, uidx, iidx, ga, gb, rg, *sems):
        wid = lax.axis_index("s") * NC + lax.axis_index("c")
        base = wid * b_per_w
        pltpu.sync_copy(u_ref.at[pl.ds(si * n + base, b_per_w)], uidx)
        pltpu.sync_copy(i_ref.at[pl.ds(si * n + base, b_per_w)], iidx)
        u0 = uidx.at[pl.ds(0, CHUNK)]
        u1 = uidx.at[pl.ds(CHUNK, CHUNK)]
        i0 = iidx.at[pl.ds(0, CHUNK)]
        i1 = iidx.at[pl.ds(CHUNK, CHUNK)]
        gh = [pltpu.async_copy(ug_ref.at[u0], ga.at[0], sems[0]),
              pltpu.async_copy(ig_ref.at[i0], gb.at[0], sems[1]),
              pltpu.async_copy(ug_ref.at[u1], ga.at[1], sems[2]),
              pltpu.async_copy(ig_ref.at[i1], gb.at[1], sems[3])]
        mh = [pltpu.async_copy(um_ref.at[u0], rg.at[0], sems[4]),
              pltpu.async_copy(im_ref.at[i0], rg.at[1], sems[5]),
              pltpu.async_copy(um_ref.at[u1], rg.at[2], sems[6])]

        def product(c):
            def body(r, _):
                for kk in range(EMBED // 16):
                    sl = pl.ds(kk * 16, 16)
                    ga[c, r, sl] = ga[c, r, sl] * gb[c, r, sl]
                return 0
            lax.fori_loop(0, CHUNK, body, 0)

        sh = []
        gh[0].wait()
        gh[1].wait()
        product(0)
        sh.append(pltpu.async_copy(
            ga.at[0], o_gp.at[pl.ds(base, CHUNK)], sems[7]))
        mh.append(pltpu.async_copy(im_ref.at[i1], gb.at[0], sems[8]))
        mh[0].wait()
        sh.append(pltpu.async_copy(
            rg.at[0], o_um.at[pl.ds(base, CHUNK)], sems[9]))
        gh[2].wait()
        gh[3].wait()
        product(1)
        sh.append(pltpu.async_copy(
            ga.at[1], o_gp.at[pl.ds(base + CHUNK, CHUNK)], sems[10]))
        mh[1].wait()
        sh.append(pltpu.async_copy(
            rg.at[1], o_im.at[pl.ds(base, CHUNK)], sems[11]))
        mh[2].wait()
        sh.append(pltpu.async_copy(
            rg.at[2], o_um.at[pl.ds(base + CHUNK, CHUNK)], sems[12]))
        mh[3].wait()
        sh.append(pltpu.async_copy(
            gb.at[0], o_im.at[pl.ds(base + CHUNK, CHUNK)], sems[13]))
        for h in sh:
            h.wait()

    return k(users, items, t_ug, t_ig, t_um, t_im)


BLK = 2048


def _dense_body(gp, um, im, w1, b1, w2, b2, pwg, pwm, out):
    h = jnp.maximum(
        um[:] @ w1[0:EMBED, :] + im[:] @ w1[EMBED:2 * EMBED, :] + b1[:], 0.0)
    m = jnp.maximum(h @ w2[:] + b2[:], 0.0)
    out[:] = (jnp.sum(gp[:] * pwg[:], axis=1)
              + jnp.sum(m * pwm[:], axis=1))


def _tc_dense(gp, um, im, W1, b1, W2, b2, pwg, pwm):
    n = gp.shape[0]
    grid = (n // BLK,)
    row_spec = pl.BlockSpec((BLK, EMBED), lambda i: (i, 0))
    full = lambda shape: pl.BlockSpec(shape, lambda i: (0,) * len(shape))
    return pl.pallas_call(
        _dense_body,
        grid=grid,
        in_specs=[
            row_spec, row_spec, row_spec,
            full((2 * EMBED, EMBED)), full((1, EMBED)),
            full((EMBED, 64)), full((1, 64)),
            full((1, EMBED)), full((1, 64)),
        ],
        out_specs=pl.BlockSpec((BLK,), lambda i: (i,)),
        out_shape=jax.ShapeDtypeStruct((n,), jnp.float32),
    )(gp, um, im, W1, b1, W2, b2, pwg, pwm)


def kernel(users, items, user_emb_gmf, item_emb_gmf, user_emb_mlp,
           item_emb_mlp, W1, b1, W2, b2, proj_w):
    users = users.astype(jnp.int32)
    items = items.astype(jnp.int32)
    b1r = b1.reshape(1, EMBED)
    b2r = b2.reshape(1, 64)
    pwg = proj_w[:EMBED].reshape(1, EMBED)
    pwm = proj_w[EMBED:].reshape(1, 64)
    n = BATCH // NSPLIT
    scores = []
    for si in range(NSPLIT):
        gp, um, im = _sc_gather4(users, items, user_emb_gmf, item_emb_gmf,
                                 user_emb_mlp, item_emb_mlp, si, n)
        scores.append(_tc_dense(gp, um, im, W1, b1r, W2, b2r, pwg, pwm))
    return jnp.concatenate(scores)


# speedup vs baseline: 1.0760x; 1.0760x over previous
import functools

import jax
import jax.numpy as jnp
from jax import lax
from jax.experimental import pallas as pl
from jax.experimental.pallas import tpu as pltpu
from jax.experimental.pallas import tpu_sc as plsc

BATCH = 16384
EMBED = 128
NC, NS = 2, 16
NW = NC * NS
CHUNK = 128
NSPLIT = 2


def _sc_gather4(users, items, t_ug, t_ig, t_um, t_im, si, n):
    b_per_w = n // NW
    assert b_per_w == 2 * CHUNK
    mesh = plsc.VectorSubcoreMesh(core_axis_name="c", subcore_axis_name="s")
    row_t = jax.ShapeDtypeStruct((n, EMBED), jnp.float32)

    @functools.partial(
        pl.kernel,
        mesh=mesh,
        out_type=(row_t, row_t, row_t),
        scratch_types=[
            pltpu.VMEM((b_per_w,), jnp.int32),
            pltpu.VMEM((b_per_w,), jnp.int32),
            pltpu.VMEM((2, CHUNK, EMBED), jnp.float32),
            pltpu.VMEM((2, CHUNK, EMBED), jnp.float32),
            pltpu.VMEM((3, CHUNK, EMBED), jnp.float32),
        ] + [pltpu.SemaphoreType.DMA] * 14,
    )
    def k(u_ref, i_ref, ug_ref, ig_ref, um_ref, im_ref,
          o_gp, o_um, o_im, uidx, iidx, ga, gb, rg, *sems):
        wid = lax.axis_index("s") * NC + lax.axis_index("c")
        base = wid * b_per_w
        pltpu.sync_copy(u_ref.at[pl.ds(si * n + base, b_per_w)], uidx)
        pltpu.sync_copy(i_ref.at[pl.ds(si * n + base, b_per_w)], iidx)
        u0 = uidx.at[pl.ds(0, CHUNK)]
        u1 = uidx.at[pl.ds(CHUNK, CHUNK)]
        i0 = iidx.at[pl.ds(0, CHUNK)]
        i1 = iidx.at[pl.ds(CHUNK, CHUNK)]
        gh = [pltpu.async_copy(ug_ref.at[u0], ga.at[0], sems[0]),
              pltpu.async_copy(ig_ref.at[i0], gb.at[0], sems[1]),
              pltpu.async_copy(ug_ref.at[u1], ga.at[1], sems[2]),
              pltpu.async_copy(ig_ref.at[i1], gb.at[1], sems[3])]
        mh = [pltpu.async_copy(um_ref.at[u0], rg.at[0], sems[4]),
              pltpu.async_copy(im_ref.at[i0], rg.at[1], sems[5]),
              pltpu.async_copy(um_ref.at[u1], rg.at[2], sems[6])]

        def product(c):
            def body(r, _):
                for kk in range(EMBED // 16):
                    sl = pl.ds(kk * 16, 16)
                    ga[c, r, sl] = ga[c, r, sl] * gb[c, r, sl]
                return 0
            lax.fori_loop(0, CHUNK, body, 0)

        sh = []
        gh[0].wait()
        gh[1].wait()
        product(0)
        sh.append(pltpu.async_copy(
            ga.at[0], o_gp.at[pl.ds(base, CHUNK)], sems[7]))
        mh.append(pltpu.async_copy(im_ref.at[i1], gb.at[0], sems[8]))
        mh[0].wait()
        sh.append(pltpu.async_copy(
            rg.at[0], o_um.at[pl.ds(base, CHUNK)], sems[9]))
        gh[2].wait()
        gh[3].wait()
        product(1)
        sh.append(pltpu.async_copy(
            ga.at[1], o_gp.at[pl.ds(base + CHUNK, CHUNK)], sems[10]))
        mh[1].wait()
        sh.append(pltpu.async_copy(
            rg.at[1], o_im.at[pl.ds(base, CHUNK)], sems[11]))
        mh[2].wait()
        sh.append(pltpu.async_copy(
            rg.at[2], o_um.at[pl.ds(base + CHUNK, CHUNK)], sems[12]))
        mh[3].wait()
        sh.append(pltpu.async_copy(
            gb.at[0], o_im.at[pl.ds(base + CHUNK, CHUNK)], sems[13]))
        for h in sh:
            h.wait()

    return k(users, items, t_ug, t_ig, t_um, t_im)


BLK = 4096


def _dense_body(gp, um, im, w1, b1, w2, b2, pwg, pwm, out):
    h = jnp.maximum(
        um[:] @ w1[0:EMBED, :] + im[:] @ w1[EMBED:2 * EMBED, :] + b1[:], 0.0)
    m = jnp.maximum(h @ w2[:] + b2[:], 0.0)
    out[:] = (jnp.sum(gp[:] * pwg[:], axis=1)
              + jnp.sum(m * pwm[:], axis=1))


def _tc_dense(gp, um, im, W1, b1, W2, b2, pwg, pwm):
    n = gp.shape[0]
    grid = (n // BLK,)
    row_spec = pl.BlockSpec((BLK, EMBED), lambda i: (i, 0))
    full = lambda shape: pl.BlockSpec(shape, lambda i: (0,) * len(shape))
    return pl.pallas_call(
        _dense_body,
        grid=grid,
        in_specs=[
            row_spec, row_spec, row_spec,
            full((2 * EMBED, EMBED)), full((1, EMBED)),
            full((EMBED, 64)), full((1, 64)),
            full((1, EMBED)), full((1, 64)),
        ],
        out_specs=pl.BlockSpec((BLK,), lambda i: (i,)),
        out_shape=jax.ShapeDtypeStruct((n,), jnp.float32),
    )(gp, um, im, W1, b1, W2, b2, pwg, pwm)


def kernel(users, items, user_emb_gmf, item_emb_gmf, user_emb_mlp,
           item_emb_mlp, W1, b1, W2, b2, proj_w):
    users = users.astype(jnp.int32)
    items = items.astype(jnp.int32)
    b1r = b1.reshape(1, EMBED)
    b2r = b2.reshape(1, 64)
    pwg = proj_w[:EMBED].reshape(1, EMBED)
    pwm = proj_w[EMBED:].reshape(1, 64)
    n = BATCH // NSPLIT
    scores = []
    for si in range(NSPLIT):
        gp, um, im = _sc_gather4(users, items, user_emb_gmf, item_emb_gmf,
                                 user_emb_mlp, item_emb_mlp, si, n)
        scores.append(_tc_dense(gp, um, im, W1, b1r, W2, b2r, pwg, pwm))
    return jnp.concatenate(scores)


# speedup vs baseline: 1.0766x; 1.0006x over previous
import functools

import jax
import jax.numpy as jnp
from jax import lax
from jax.experimental import pallas as pl
from jax.experimental.pallas import tpu as pltpu
from jax.experimental.pallas import tpu_sc as plsc

BATCH = 16384
EMBED = 128
NC, NS = 2, 16
NW = NC * NS
CHUNK = 128
NSPLIT = 2


def _sc_gather4(users, items, t_ug, t_ig, t_um, t_im, si, n):
    b_per_w = n // NW
    assert b_per_w == 2 * CHUNK
    mesh = plsc.VectorSubcoreMesh(core_axis_name="c", subcore_axis_name="s")
    row_t = jax.ShapeDtypeStruct((n, EMBED), jnp.float32)

    @functools.partial(
        pl.kernel,
        mesh=mesh,
        out_type=(row_t, row_t, row_t),
        scratch_types=[
            pltpu.VMEM((b_per_w,), jnp.int32),
            pltpu.VMEM((b_per_w,), jnp.int32),
            pltpu.VMEM((2, CHUNK, EMBED), jnp.float32),
            pltpu.VMEM((2, CHUNK, EMBED), jnp.float32),
            pltpu.VMEM((3, CHUNK, EMBED), jnp.float32),
        ] + [pltpu.SemaphoreType.DMA] * 14,
    )
    def k(u_ref, i_ref, ug_ref, ig_ref, um_ref, im_ref,
          o_gp, o_um, o_im, uidx, iidx, ga, gb, rg, *sems):
        wid = lax.axis_index("s") * NC + lax.axis_index("c")
        base = wid * b_per_w
        pltpu.sync_copy(u_ref.at[pl.ds(si * n + base, b_per_w)], uidx)
        pltpu.sync_copy(i_ref.at[pl.ds(si * n + base, b_per_w)], iidx)
        u0 = uidx.at[pl.ds(0, CHUNK)]
        u1 = uidx.at[pl.ds(CHUNK, CHUNK)]
        i0 = iidx.at[pl.ds(0, CHUNK)]
        i1 = iidx.at[pl.ds(CHUNK, CHUNK)]
        gh = [pltpu.async_copy(ug_ref.at[u0], ga.at[0], sems[0]),
              pltpu.async_copy(ig_ref.at[i0], gb.at[0], sems[1]),
              pltpu.async_copy(ug_ref.at[u1], ga.at[1], sems[2]),
              pltpu.async_copy(ig_ref.at[i1], gb.at[1], sems[3])]
        mh = [pltpu.async_copy(um_ref.at[u0], rg.at[0], sems[4]),
              pltpu.async_copy(im_ref.at[i0], rg.at[1], sems[5]),
              pltpu.async_copy(um_ref.at[u1], rg.at[2], sems[6])]

        def product(c):
            def body(r, _):
                for kk in range(EMBED // 16):
                    sl = pl.ds(kk * 16, 16)
                    ga[c, r, sl] = ga[c, r, sl] * gb[c, r, sl]
                return 0
            lax.fori_loop(0, CHUNK, body, 0)

        sh = []
        gh[0].wait()
        gh[1].wait()
        product(0)
        sh.append(pltpu.async_copy(
            ga.at[0], o_gp.at[pl.ds(base, CHUNK)], sems[7]))
        mh.append(pltpu.async_copy(im_ref.at[i1], gb.at[0], sems[8]))
        mh[0].wait()
        sh.append(pltpu.async_copy(
            rg.at[0], o_um.at[pl.ds(base, CHUNK)], sems[9]))
        gh[2].wait()
        gh[3].wait()
        product(1)
        sh.append(pltpu.async_copy(
            ga.at[1], o_gp.at[pl.ds(base + CHUNK, CHUNK)], sems[10]))
        mh[1].wait()
        sh.append(pltpu.async_copy(
            rg.at[1], o_im.at[pl.ds(base, CHUNK)], sems[11]))
        mh[2].wait()
        sh.append(pltpu.async_copy(
            rg.at[2], o_um.at[pl.ds(base + CHUNK, CHUNK)], sems[12]))
        mh[3].wait()
        sh.append(pltpu.async_copy(
            gb.at[0], o_im.at[pl.ds(base + CHUNK, CHUNK)], sems[13]))
        for h in sh:
            h.wait()

    return k(users, items, t_ug, t_ig, t_um, t_im)


BLK = 8192


def _dense_body(gp, um, im, w1, b1, w2, b2, pwg, pwm, out):
    h = jnp.maximum(
        um[:] @ w1[0:EMBED, :] + im[:] @ w1[EMBED:2 * EMBED, :] + b1[:], 0.0)
    m = jnp.maximum(h @ w2[:] + b2[:], 0.0)
    out[:] = (jnp.sum(gp[:] * pwg[:], axis=1)
              + jnp.sum(m * pwm[:], axis=1))


def _tc_dense(gp, um, im, W1, b1, W2, b2, pwg, pwm):
    n = gp.shape[0]
    grid = (n // BLK,)
    row_spec = pl.BlockSpec((BLK, EMBED), lambda i: (i, 0))
    full = lambda shape: pl.BlockSpec(shape, lambda i: (0,) * len(shape))
    return pl.pallas_call(
        _dense_body,
        grid=grid,
        in_specs=[
            row_spec, row_spec, row_spec,
            full((2 * EMBED, EMBED)), full((1, EMBED)),
            full((EMBED, 64)), full((1, 64)),
            full((1, EMBED)), full((1, 64)),
        ],
        out_specs=pl.BlockSpec((BLK,), lambda i: (i,)),
        out_shape=jax.ShapeDtypeStruct((n,), jnp.float32),
    )(gp, um, im, W1, b1, W2, b2, pwg, pwm)


def kernel(users, items, user_emb_gmf, item_emb_gmf, user_emb_mlp,
           item_emb_mlp, W1, b1, W2, b2, proj_w):
    users = users.astype(jnp.int32)
    items = items.astype(jnp.int32)
    b1r = b1.reshape(1, EMBED)
    b2r = b2.reshape(1, 64)
    pwg = proj_w[:EMBED].reshape(1, EMBED)
    pwm = proj_w[EMBED:].reshape(1, 64)
    n = BATCH // NSPLIT
    scores = []
    for si in range(NSPLIT):
        gp, um, im = _sc_gather4(users, items, user_emb_gmf, item_emb_gmf,
                                 user_emb_mlp, item_emb_mlp, si, n)
        scores.append(_tc_dense(gp, um, im, W1, b1r, W2, b2r, pwg, pwm))
    return jnp.concatenate(scores)
